# baseline (speedup 1.0000x reference)
import functools
import jax
import jax.numpy as jnp
from jax import lax
from jax.experimental import pallas as pl
from jax.experimental.pallas import tpu as pltpu
from jax.experimental.pallas import tpu_sc as plsc

B, N, D = 128, 8192, 64
NW = 32
RPW = B // NW
C = 256
NT = 5632
NS = N - NT
NCH = NS // C
GRP = C // 16


def _sc_body(x_hbm, val_hbm, idx_hbm, buf0, buf1, vbuf, ibuf, sem0, sem1):
    cid = lax.axis_index("c")
    sid = lax.axis_index("s")
    wid = sid * 2 + cid
    iota = lax.iota(jnp.int32, 16)

    def compute_chunk(buf, cap0, carry):
        def group(g, carry):
            bv, bi = carry
            cap = g * 16 + iota
            z = jnp.zeros((16,), jnp.float32)
            accs = [z] * 8
            dv = iota
            for d in range(D):
                val = plsc.load_gather(buf, [cap, dv])
                accs[d % 8] = accs[d % 8] + val * val
                if d < D - 1:
                    dv = dv + 1
                    if d + 1 > D - 16:
                        dv = dv & (D - 1)
            s = ((accs[0] + accs[1]) + (accs[2] + accs[3])) + (
                (accs[4] + accs[5]) + (accs[6] + accs[7])
            )
            gcap = cap0 + cap
            better = s > bv
            return jnp.where(better, s, bv), jnp.where(better, gcap, bi)

        return lax.fori_loop(0, GRP, group, carry)

    vvec = jnp.full((16,), -1.0, jnp.float32)
    ivec = jnp.zeros((16,), jnp.int32)
    for j in range(RPW):
        b = wid * RPW + j
        pltpu.async_copy(x_hbm.at[b, pl.ds(NT, C)], buf0, sem0)
        pltpu.async_copy(x_hbm.at[b, pl.ds(NT + C, C)], buf1, sem1)

        def chunk_pair(i, carry, b=b):
            pltpu.make_async_copy(x_hbm.at[b, pl.ds(NT, C)], buf0, sem0).wait()
            carry = compute_chunk(buf0, NT + 2 * i * C, carry)

            @pl.when(i < NCH // 2 - 1)
            def _():
                pltpu.async_copy(
                    x_hbm.at[b, pl.ds(NT + (2 * i + 2) * C, C)], buf0, sem0
                )

            pltpu.make_async_copy(x_hbm.at[b, pl.ds(NT, C)], buf1, sem1).wait()
            carry = compute_chunk(buf1, NT + (2 * i + 1) * C, carry)

            @pl.when(i < NCH // 2 - 1)
            def _():
                pltpu.async_copy(
                    x_hbm.at[b, pl.ds(NT + (2 * i + 3) * C, C)], buf1, sem1
                )

            return carry

        init = (jnp.full((16,), -1.0, jnp.float32), jnp.zeros((16,), jnp.int32))
        bv, bi = lax.fori_loop(0, NCH // 2, chunk_pair, init)

        m = jnp.max(bv)
        cand = jnp.where(bv == m, bi, jnp.int32(1 << 30))
        win = jnp.min(cand)
        lane_j = iota == j
        vvec = jnp.where(lane_j, jnp.full((16,), 1.0, jnp.float32) * m, vvec)
        ivec = jnp.where(lane_j, jnp.full((16,), 1, jnp.int32) * win, ivec)

    vbuf[...] = vvec
    ibuf[...] = ivec
    pltpu.sync_copy(vbuf, val_hbm.at[pl.ds(wid * 16, 16)])
    pltpu.sync_copy(ibuf, idx_hbm.at[pl.ds(wid * 16, 16)])


_sc_kernel = functools.partial(
    pl.kernel,
    mesh=plsc.VectorSubcoreMesh(core_axis_name="c", subcore_axis_name="s"),
    compiler_params=pltpu.CompilerParams(needs_layout_passes=False),
    out_type=(
        jax.ShapeDtypeStruct((NW * 16,), jnp.float32),
        jax.ShapeDtypeStruct((NW * 16,), jnp.int32),
    ),
    scratch_types=[
        pltpu.VMEM((C, D), jnp.float32),
        pltpu.VMEM((C, D), jnp.float32),
        pltpu.VMEM((16,), jnp.float32),
        pltpu.VMEM((16,), jnp.int32),
        pltpu.SemaphoreType.DMA,
        pltpu.SemaphoreType.DMA,
    ],
)(_sc_body)


def _tc_body(x_ref, val_ref, idx_ref):
    x = x_ref[0]
    s = jnp.sum(x * x, axis=1)
    m = jnp.max(s)
    iota = lax.broadcasted_iota(jnp.int32, (NT,), 0)
    i = jnp.min(jnp.where(s == m, iota, NT))
    val_ref[0, 0, 0] = m
    idx_ref[0, 0, 0] = i


def _tc_partial(x):
    return pl.pallas_call(
        _tc_body,
        grid=(B,),
        in_specs=[pl.BlockSpec((1, NT, D), lambda i: (i, 0, 0))],
        out_specs=(
            pl.BlockSpec((1, 1, 1), lambda i: (i, 0, 0), memory_space=pltpu.SMEM),
            pl.BlockSpec((1, 1, 1), lambda i: (i, 0, 0), memory_space=pltpu.SMEM),
        ),
        out_shape=(
            jax.ShapeDtypeStruct((B, 1, 1), jnp.float32),
            jax.ShapeDtypeStruct((B, 1, 1), jnp.int32),
        ),
    )(x)


def _merge_body(vt_ref, it_ref, vs_ref, is_ref, x_ref, o_ref, sem):
    idxs = []
    for b in range(B):
        vt = vt_ref[b, 0, 0]
        it = it_ref[b, 0, 0]
        vs = vs_ref[b]
        isx = is_ref[b]
        i = jnp.where(vs > vt, isx, it)
        idxs.append(i)
        pltpu.make_async_copy(
            x_ref.at[b, pl.ds(i, 1), :], o_ref.at[pl.ds(b, 1)], sem
        ).start()
    for b in range(B):
        pltpu.make_async_copy(
            x_ref.at[b, pl.ds(idxs[b], 1), :], o_ref.at[pl.ds(b, 1)], sem
        ).wait()


def _merge(vt, it, vs, isx, x):
    return pl.pallas_call(
        _merge_body,
        in_specs=[
            pl.BlockSpec(memory_space=pltpu.SMEM),
            pl.BlockSpec(memory_space=pltpu.SMEM),
            pl.BlockSpec(memory_space=pltpu.SMEM),
            pl.BlockSpec(memory_space=pltpu.SMEM),
            pl.BlockSpec(memory_space=pl.ANY),
        ],
        out_specs=pl.BlockSpec(memory_space=pl.ANY),
        out_shape=jax.ShapeDtypeStruct((B, D), jnp.float32),
        scratch_shapes=[pltpu.SemaphoreType.DMA],
    )(vt, it, vs, isx, x)


def kernel(input):
    vs, isx = _sc_kernel(input)
    vs = vs.reshape(NW, 16)[:, :RPW].reshape(B)
    isx = isx.reshape(NW, 16)[:, :RPW].reshape(B)
    vt, it = _tc_partial(input)
    return _merge(vt, it, vs, isx, input)


# speedup vs baseline: 1.2300x; 1.2300x over previous
import functools
import jax
import jax.numpy as jnp
from jax import lax
from jax.experimental import pallas as pl
from jax.experimental.pallas import tpu as pltpu
from jax.experimental.pallas import tpu_sc as plsc

B, N, D = 128, 8192, 64
NW = 32
RPW = B // NW
C = 256
NT = 5632
NS = N - NT
NCH = NS // C
GRP = C // 16


def _sc_body(x_hbm, val_hbm, idx_hbm, buf0, buf1, vbuf, ibuf, sem0, sem1):
    cid = lax.axis_index("c")
    sid = lax.axis_index("s")
    wid = sid * 2 + cid
    iota = lax.iota(jnp.int32, 16)

    def compute_chunk(buf, cap0, carry):
        def group(g, carry):
            bv, bi = carry
            cap = g * 16 + iota
            z = jnp.zeros((16,), jnp.float32)
            accs = [z] * 8
            dv = iota
            for d in range(D):
                val = plsc.load_gather(buf, [cap, dv])
                accs[d % 8] = accs[d % 8] + val * val
                if d < D - 1:
                    dv = dv + 1
                    if d + 1 > D - 16:
                        dv = dv & (D - 1)
            s = ((accs[0] + accs[1]) + (accs[2] + accs[3])) + (
                (accs[4] + accs[5]) + (accs[6] + accs[7])
            )
            gcap = cap0 + cap
            better = s > bv
            return jnp.where(better, s, bv), jnp.where(better, gcap, bi)

        return lax.fori_loop(0, GRP, group, carry)

    vvec = jnp.full((16,), -1.0, jnp.float32)
    ivec = jnp.zeros((16,), jnp.int32)
    for j in range(RPW):
        b = wid * RPW + j
        pltpu.async_copy(x_hbm.at[b, pl.ds(NT, C)], buf0, sem0)
        pltpu.async_copy(x_hbm.at[b, pl.ds(NT + C, C)], buf1, sem1)

        def chunk_pair(i, carry, b=b):
            pltpu.make_async_copy(x_hbm.at[b, pl.ds(NT, C)], buf0, sem0).wait()
            carry = compute_chunk(buf0, NT + 2 * i * C, carry)

            @pl.when(i < NCH // 2 - 1)
            def _():
                pltpu.async_copy(
                    x_hbm.at[b, pl.ds(NT + (2 * i + 2) * C, C)], buf0, sem0
                )

            pltpu.make_async_copy(x_hbm.at[b, pl.ds(NT, C)], buf1, sem1).wait()
            carry = compute_chunk(buf1, NT + (2 * i + 1) * C, carry)

            @pl.when(i < NCH // 2 - 1)
            def _():
                pltpu.async_copy(
                    x_hbm.at[b, pl.ds(NT + (2 * i + 3) * C, C)], buf1, sem1
                )

            return carry

        init = (jnp.full((16,), -1.0, jnp.float32), jnp.zeros((16,), jnp.int32))
        bv, bi = lax.fori_loop(0, NCH // 2, chunk_pair, init)

        m = jnp.max(bv)
        cand = jnp.where(bv == m, bi, jnp.int32(1 << 30))
        win = jnp.min(cand)
        lane_j = iota == j
        vvec = jnp.where(lane_j, jnp.full((16,), 1.0, jnp.float32) * m, vvec)
        ivec = jnp.where(lane_j, jnp.full((16,), 1, jnp.int32) * win, ivec)

    vbuf[...] = vvec
    ibuf[...] = ivec
    pltpu.sync_copy(vbuf, val_hbm.at[pl.ds(wid * 16, 16)])
    pltpu.sync_copy(ibuf, idx_hbm.at[pl.ds(wid * 16, 16)])


_sc_kernel = functools.partial(
    pl.kernel,
    mesh=plsc.VectorSubcoreMesh(core_axis_name="c", subcore_axis_name="s"),
    compiler_params=pltpu.CompilerParams(needs_layout_passes=False),
    out_type=(
        jax.ShapeDtypeStruct((NW * 16,), jnp.float32),
        jax.ShapeDtypeStruct((NW * 16,), jnp.int32),
    ),
    scratch_types=[
        pltpu.VMEM((C, D), jnp.float32),
        pltpu.VMEM((C, D), jnp.float32),
        pltpu.VMEM((16,), jnp.float32),
        pltpu.VMEM((16,), jnp.int32),
        pltpu.SemaphoreType.DMA,
        pltpu.SemaphoreType.DMA,
    ],
)(_sc_body)


def _tc_body(x_ref, val_ref, idx_ref):
    x = x_ref[0]
    s = jnp.sum(x * x, axis=1)
    i = jnp.argmax(s)
    val_ref[0, 0, 0] = jnp.max(s)
    idx_ref[0, 0, 0] = i


def _tc_partial(x):
    return pl.pallas_call(
        _tc_body,
        grid=(B,),
        in_specs=[pl.BlockSpec((1, NT, D), lambda i: (i, 0, 0))],
        out_specs=(
            pl.BlockSpec((1, 1, 1), lambda i: (i, 0, 0), memory_space=pltpu.SMEM),
            pl.BlockSpec((1, 1, 1), lambda i: (i, 0, 0), memory_space=pltpu.SMEM),
        ),
        out_shape=(
            jax.ShapeDtypeStruct((B, 1, 1), jnp.float32),
            jax.ShapeDtypeStruct((B, 1, 1), jnp.int32),
        ),
    )(x)


def _merge_body(vt_ref, it_ref, vs_ref, is_ref, x_ref, o_ref, sem):
    idxs = []
    for b in range(B):
        vt = vt_ref[b, 0, 0]
        it = it_ref[b, 0, 0]
        vs = vs_ref[b]
        isx = is_ref[b]
        i = jnp.where(vs > vt, isx, it)
        idxs.append(i)
        pltpu.make_async_copy(
            x_ref.at[b, pl.ds(i, 1), :], o_ref.at[pl.ds(b, 1)], sem
        ).start()
    for b in range(B):
        pltpu.make_async_copy(
            x_ref.at[b, pl.ds(idxs[b], 1), :], o_ref.at[pl.ds(b, 1)], sem
        ).wait()


def _merge(vt, it, vs, isx, x):
    return pl.pallas_call(
        _merge_body,
        in_specs=[
            pl.BlockSpec(memory_space=pltpu.SMEM),
            pl.BlockSpec(memory_space=pltpu.SMEM),
            pl.BlockSpec(memory_space=pltpu.SMEM),
            pl.BlockSpec(memory_space=pltpu.SMEM),
            pl.BlockSpec(memory_space=pl.ANY),
        ],
        out_specs=pl.BlockSpec(memory_space=pl.ANY),
        out_shape=jax.ShapeDtypeStruct((B, D), jnp.float32),
        scratch_shapes=[pltpu.SemaphoreType.DMA],
    )(vt, it, vs, isx, x)


def kernel(input):
    vs, isx = _sc_kernel(input)
    vs = vs.reshape(NW, 16)[:, :RPW].reshape(B)
    isx = isx.reshape(NW, 16)[:, :RPW].reshape(B)
    vt, it = _tc_partial(input)
    return _merge(vt, it, vs, isx, input)
